# Initial kernel scaffold; baseline (speedup 1.0000x reference)
import jax
import jax.numpy as jnp
from jax.experimental import pallas as pl


def kernel(five_token_indices, table, W1, b1, W2, b2):
    raise NotImplementedError("write your pallas kernel here")


# speedup vs baseline: 1.5977x; 1.5977x over previous
import jax
import jax.numpy as jnp
from jax import lax
from jax.experimental import pallas as pl
from jax.experimental.pallas import tpu as pltpu
from jax.experimental.pallas import tpu_sc as plsc

VOCAB = 1000000
D = 50
DP = 56
BATCH = 16384
IN_DIM = DP * 5
HIDDEN = 128
TAGS = 45

TOKENS = BATCH * 5
NC = 2
NS = 16
NW = NC * NS
R_PER_W = TOKENS // NW
CH = 128
N_CH = R_PER_W // CH


def _sc_gather_kernel(idx_hbm, table_hbm, out_hbm, idx_v, rows_v, sem):
    wid = lax.axis_index("s") * NC + lax.axis_index("c")
    base = wid * R_PER_W
    pltpu.sync_copy(idx_hbm.at[wid], idx_v)

    def chunk(ci, carry):
        pltpu.async_copy(
            table_hbm.at[idx_v.at[ci]], rows_v, sem
        ).wait()
        pltpu.sync_copy(rows_v, out_hbm.at[pl.ds(base + ci * CH, CH)])
        return carry

    lax.fori_loop(0, N_CH, chunk, 0)


def _sc_gather(idx, table_p):
    mesh = plsc.VectorSubcoreMesh(core_axis_name="c", subcore_axis_name="s")
    return pl.kernel(
        _sc_gather_kernel,
        mesh=mesh,
        out_type=jax.ShapeDtypeStruct((TOKENS, DP), jnp.float32),
        scratch_types=[
            pltpu.VMEM((N_CH, CH), jnp.int32),
            pltpu.VMEM((CH, DP), jnp.float32),
            pltpu.SemaphoreType.DMA,
        ],
        compiler_params=pltpu.CompilerParams(use_tc_tiling_on_sc=False),
    )(idx.reshape(NW, N_CH, CH), table_p)


def _mlp_kernel(x_ref, w1_ref, b1_ref, w2_ref, b2_ref, o_ref):
    h = jnp.tanh(
        jnp.dot(x_ref[...], w1_ref[...], preferred_element_type=jnp.float32)
        + b1_ref[...]
    )
    o_ref[...] = (
        jnp.dot(h, w2_ref[...], preferred_element_type=jnp.float32)
        + b2_ref[...]
    )


def _tc_mlp(x, W1p, b1, W2, b2):
    blk = 2048
    grid = (BATCH // blk,)
    return pl.pallas_call(
        _mlp_kernel,
        grid=grid,
        in_specs=[
            pl.BlockSpec((blk, IN_DIM), lambda i: (i, 0)),
            pl.BlockSpec((IN_DIM, HIDDEN), lambda i: (0, 0)),
            pl.BlockSpec((1, HIDDEN), lambda i: (0, 0)),
            pl.BlockSpec((HIDDEN, TAGS), lambda i: (0, 0)),
            pl.BlockSpec((1, TAGS), lambda i: (0, 0)),
        ],
        out_specs=pl.BlockSpec((blk, TAGS), lambda i: (i, 0)),
        out_shape=jax.ShapeDtypeStruct((BATCH, TAGS), jnp.float32),
    )(x, W1p, b1.reshape(1, HIDDEN), W2, b2.reshape(1, TAGS))


def kernel(five_token_indices, table, W1, b1, W2, b2):
    idx = five_token_indices.reshape(-1).astype(jnp.int32)
    table_p = jnp.pad(table, ((0, 0), (0, DP - D)))
    W1p = jnp.pad(W1.reshape(5, D, HIDDEN), ((0, 0), (0, DP - D), (0, 0)))
    W1p = W1p.reshape(IN_DIM, HIDDEN)
    rows = _sc_gather(idx, table_p)
    x = rows.reshape(BATCH, IN_DIM)
    return _tc_mlp(x, W1p, b1, W2, b2)


# speedup vs baseline: 5.8478x; 3.6602x over previous
import jax
import jax.numpy as jnp
from jax import lax
from jax.experimental import pallas as pl
from jax.experimental.pallas import tpu as pltpu
from jax.experimental.pallas import tpu_sc as plsc

VOCAB = 1000000
D = 50
DP = 128
BATCH = 16384
IN_DIM = DP * 5
HIDDEN = 128
TAGS = 45

TOKENS = BATCH * 5
NC = 2
NS = 16
NW = NC * NS
R_PER_W = TOKENS // NW
CH = 128
N_CH = R_PER_W // CH

TBLK = 4096


def _repack_kernel(t_ref, o_ref):
    xt = t_ref[...].T
    o_ref[...] = jnp.concatenate(
        [xt, jnp.zeros((TBLK, DP - D), jnp.float32)], axis=1
    )


def _tc_repack(tableT):
    grid = (pl.cdiv(VOCAB, TBLK),)
    return pl.pallas_call(
        _repack_kernel,
        grid=grid,
        in_specs=[pl.BlockSpec((D, TBLK), lambda i: (0, i))],
        out_specs=pl.BlockSpec((TBLK, DP), lambda i: (i, 0)),
        out_shape=jax.ShapeDtypeStruct((VOCAB, DP), jnp.float32),
    )(tableT)


def _sc_gather_kernel(idx_hbm, table_hbm, out_hbm, idx_v, rows_v, sem):
    wid = lax.axis_index("s") * NC + lax.axis_index("c")
    base = wid * R_PER_W
    pltpu.sync_copy(idx_hbm.at[wid], idx_v)

    def chunk(ci, carry):
        pltpu.async_copy(
            table_hbm.at[idx_v.at[ci]], rows_v, sem
        ).wait()
        pltpu.sync_copy(rows_v, out_hbm.at[pl.ds(base + ci * CH, CH)])
        return carry

    lax.fori_loop(0, N_CH, chunk, 0)


def _sc_gather(idx, table_p):
    mesh = plsc.VectorSubcoreMesh(core_axis_name="c", subcore_axis_name="s")
    return pl.kernel(
        _sc_gather_kernel,
        mesh=mesh,
        out_type=jax.ShapeDtypeStruct((TOKENS, DP), jnp.float32),
        scratch_types=[
            pltpu.VMEM((N_CH, CH), jnp.int32),
            pltpu.VMEM((CH, DP), jnp.float32),
            pltpu.SemaphoreType.DMA,
        ],
    )(idx.reshape(NW, N_CH, CH), table_p)


def _mlp_kernel(x_ref, w1_ref, b1_ref, w2_ref, b2_ref, o_ref):
    h = jnp.tanh(
        jnp.dot(x_ref[...], w1_ref[...], preferred_element_type=jnp.float32)
        + b1_ref[...]
    )
    o_ref[...] = (
        jnp.dot(h, w2_ref[...], preferred_element_type=jnp.float32)
        + b2_ref[...]
    )


def _tc_mlp(x, W1p, b1, W2, b2):
    blk = 2048
    grid = (BATCH // blk,)
    return pl.pallas_call(
        _mlp_kernel,
        grid=grid,
        in_specs=[
            pl.BlockSpec((blk, IN_DIM), lambda i: (i, 0)),
            pl.BlockSpec((IN_DIM, HIDDEN), lambda i: (0, 0)),
            pl.BlockSpec((1, HIDDEN), lambda i: (0, 0)),
            pl.BlockSpec((HIDDEN, TAGS), lambda i: (0, 0)),
            pl.BlockSpec((1, TAGS), lambda i: (0, 0)),
        ],
        out_specs=pl.BlockSpec((blk, TAGS), lambda i: (i, 0)),
        out_shape=jax.ShapeDtypeStruct((BATCH, TAGS), jnp.float32),
    )(x, W1p, b1.reshape(1, HIDDEN), W2, b2.reshape(1, TAGS))


def kernel(five_token_indices, table, W1, b1, W2, b2):
    idx = five_token_indices.reshape(-1).astype(jnp.int32)
    table_p = _tc_repack(table.T)
    W1p = jnp.pad(W1.reshape(5, D, HIDDEN), ((0, 0), (0, DP - D), (0, 0)))
    W1p = W1p.reshape(IN_DIM, HIDDEN)
    rows = _sc_gather(idx, table_p)
    x = rows.reshape(BATCH, IN_DIM)
    return _tc_mlp(x, W1p, b1, W2, b2)


# speedup vs baseline: 6.5160x; 1.1143x over previous
import jax
import jax.numpy as jnp
from jax import lax
from jax.experimental import pallas as pl
from jax.experimental.pallas import tpu as pltpu
from jax.experimental.pallas import tpu_sc as plsc

VOCAB = 1000000
D = 50
DP = 64
BATCH = 16384
IN_DIM = DP * 5
HIDDEN = 128
TAGS = 45

TOKENS = BATCH * 5
NC = 2
NS = 16
NW = NC * NS
R_PER_W = TOKENS // NW
CH = 128
N_CH = R_PER_W // CH

TBLK = 4096
HBLK = TBLK // 2
NBLK = pl.cdiv(VOCAB, TBLK)
PROWS = NBLK * TBLK


def _repack_kernel(t_ref, o_ref):
    xt = t_ref[...].T
    xp = jnp.concatenate(
        [xt, jnp.zeros((TBLK, DP - D), jnp.float32)], axis=1
    )
    o_ref[...] = jnp.concatenate([xp[:HBLK], xp[HBLK:]], axis=1)


def _tc_repack(tableT):
    return pl.pallas_call(
        _repack_kernel,
        grid=(NBLK,),
        in_specs=[pl.BlockSpec((D, TBLK), lambda i: (0, i))],
        out_specs=pl.BlockSpec((HBLK, 2 * DP), lambda i: (i, 0)),
        out_shape=jax.ShapeDtypeStruct((NBLK * HBLK, 2 * DP), jnp.float32),
    )(tableT)


def _sc_gather_kernel(idx_hbm, table_hbm, out_hbm, idx_v, rows_v, sem):
    wid = lax.axis_index("s") * NC + lax.axis_index("c")
    base = wid * R_PER_W
    pltpu.sync_copy(idx_hbm.at[wid], idx_v)

    def chunk(ci, carry):
        pltpu.async_copy(
            table_hbm.at[idx_v.at[ci]], rows_v, sem
        ).wait()
        pltpu.sync_copy(rows_v, out_hbm.at[pl.ds(base + ci * CH, CH)])
        return carry

    lax.fori_loop(0, N_CH, chunk, 0)


def _sc_gather(idx, table_p):
    mesh = plsc.VectorSubcoreMesh(core_axis_name="c", subcore_axis_name="s")
    return pl.kernel(
        _sc_gather_kernel,
        mesh=mesh,
        out_type=jax.ShapeDtypeStruct((TOKENS, DP), jnp.float32),
        scratch_types=[
            pltpu.VMEM((N_CH, CH), jnp.int32),
            pltpu.VMEM((CH, DP), jnp.float32),
            pltpu.SemaphoreType.DMA,
        ],
        compiler_params=pltpu.CompilerParams(use_tc_tiling_on_sc=False),
    )(idx.reshape(NW, N_CH, CH), table_p)


def _packed_row(idx):
    blk = idx // TBLK
    p = idx % TBLK
    return 2 * (blk * HBLK + (p % HBLK)) + p // HBLK


def _mlp_kernel(x_ref, w1_ref, b1_ref, w2_ref, b2_ref, o_ref):
    h = jnp.tanh(
        jnp.dot(x_ref[...], w1_ref[...], preferred_element_type=jnp.float32)
        + b1_ref[...]
    )
    o_ref[...] = (
        jnp.dot(h, w2_ref[...], preferred_element_type=jnp.float32)
        + b2_ref[...]
    )


def _tc_mlp(x, W1p, b1, W2, b2):
    blk = 2048
    grid = (BATCH // blk,)
    return pl.pallas_call(
        _mlp_kernel,
        grid=grid,
        in_specs=[
            pl.BlockSpec((blk, IN_DIM), lambda i: (i, 0)),
            pl.BlockSpec((IN_DIM, HIDDEN), lambda i: (0, 0)),
            pl.BlockSpec((1, HIDDEN), lambda i: (0, 0)),
            pl.BlockSpec((HIDDEN, TAGS), lambda i: (0, 0)),
            pl.BlockSpec((1, TAGS), lambda i: (0, 0)),
        ],
        out_specs=pl.BlockSpec((blk, TAGS), lambda i: (i, 0)),
        out_shape=jax.ShapeDtypeStruct((BATCH, TAGS), jnp.float32),
    )(x, W1p, b1.reshape(1, HIDDEN), W2, b2.reshape(1, TAGS))


def kernel(five_token_indices, table, W1, b1, W2, b2):
    idx = _packed_row(five_token_indices.reshape(-1).astype(jnp.int32))
    packed = _tc_repack(table.T)
    table_p = packed.reshape(2 * NBLK * HBLK, DP)
    W1p = jnp.pad(W1.reshape(5, D, HIDDEN), ((0, 0), (0, DP - D), (0, 0)))
    W1p = W1p.reshape(IN_DIM, HIDDEN)
    rows = _sc_gather(idx, table_p)
    x = rows.reshape(BATCH, IN_DIM)
    return _tc_mlp(x, W1p, b1, W2, b2)


# speedup vs baseline: 7.7259x; 1.1857x over previous
import jax
import jax.numpy as jnp
from jax import lax
from jax.experimental import pallas as pl
from jax.experimental.pallas import tpu as pltpu
from jax.experimental.pallas import tpu_sc as plsc

VOCAB = 1000000
D = 50
DP = 64
BATCH = 16384
IN_DIM = DP * 5
HIDDEN = 128
TAGS = 45

TOKENS = BATCH * 5
NC = 2
NS = 16
NW = NC * NS
R_PER_W = TOKENS // NW
CH = 128
N_CH = R_PER_W // CH

TBLK = 8192
HBLK = TBLK // 2
NBLK = pl.cdiv(VOCAB, TBLK)
PROWS = NBLK * TBLK


def _repack_kernel(t_ref, o_ref):
    xt = t_ref[...].T
    xp = jnp.concatenate(
        [xt, jnp.zeros((TBLK, DP - D), jnp.float32)], axis=1
    )
    o_ref[...] = jnp.concatenate([xp[:HBLK], xp[HBLK:]], axis=1)


def _tc_repack(tableT):
    return pl.pallas_call(
        _repack_kernel,
        grid=(NBLK,),
        in_specs=[pl.BlockSpec((D, TBLK), lambda i: (0, i))],
        out_specs=pl.BlockSpec((HBLK, 2 * DP), lambda i: (i, 0)),
        out_shape=jax.ShapeDtypeStruct((NBLK * HBLK, 2 * DP), jnp.float32),
    )(tableT)


def _sc_gather_kernel(idx_hbm, table_hbm, out_hbm, idx_v, rows_v, sem):
    wid = lax.axis_index("s") * NC + lax.axis_index("c")
    base = wid * R_PER_W
    pltpu.sync_copy(idx_hbm.at[wid], idx_v)

    def chunk(ci, carry):
        pltpu.async_copy(
            table_hbm.at[idx_v.at[ci]], rows_v, sem
        ).wait()
        pltpu.sync_copy(rows_v, out_hbm.at[pl.ds(base + ci * CH, CH)])
        return carry

    lax.fori_loop(0, N_CH, chunk, 0)


def _sc_gather(idx, table_p):
    mesh = plsc.VectorSubcoreMesh(core_axis_name="c", subcore_axis_name="s")
    return pl.kernel(
        _sc_gather_kernel,
        mesh=mesh,
        out_type=jax.ShapeDtypeStruct((TOKENS, DP), jnp.float32),
        scratch_types=[
            pltpu.VMEM((N_CH, CH), jnp.int32),
            pltpu.VMEM((CH, DP), jnp.float32),
            pltpu.SemaphoreType.DMA,
        ],
        compiler_params=pltpu.CompilerParams(use_tc_tiling_on_sc=False),
    )(idx.reshape(NW, N_CH, CH), table_p)


def _packed_row(idx):
    blk = idx // TBLK
    p = idx % TBLK
    return 2 * (blk * HBLK + (p % HBLK)) + p // HBLK


def _mlp_kernel(x_ref, w1_ref, b1_ref, w2_ref, b2_ref, o_ref):
    h = jnp.tanh(
        jnp.dot(x_ref[...], w1_ref[...], preferred_element_type=jnp.float32)
        + b1_ref[...]
    )
    o_ref[...] = (
        jnp.dot(h, w2_ref[...], preferred_element_type=jnp.float32)
        + b2_ref[...]
    )


def _tc_mlp(x, W1p, b1, W2, b2):
    blk = 2048
    grid = (BATCH // blk,)
    return pl.pallas_call(
        _mlp_kernel,
        grid=grid,
        in_specs=[
            pl.BlockSpec((blk, IN_DIM), lambda i: (i, 0)),
            pl.BlockSpec((IN_DIM, HIDDEN), lambda i: (0, 0)),
            pl.BlockSpec((1, HIDDEN), lambda i: (0, 0)),
            pl.BlockSpec((HIDDEN, TAGS), lambda i: (0, 0)),
            pl.BlockSpec((1, TAGS), lambda i: (0, 0)),
        ],
        out_specs=pl.BlockSpec((blk, TAGS), lambda i: (i, 0)),
        out_shape=jax.ShapeDtypeStruct((BATCH, TAGS), jnp.float32),
    )(x, W1p, b1.reshape(1, HIDDEN), W2, b2.reshape(1, TAGS))


def kernel(five_token_indices, table, W1, b1, W2, b2):
    idx = _packed_row(five_token_indices.reshape(-1).astype(jnp.int32))
    packed = _tc_repack(table.T)
    table_p = packed.reshape(2 * NBLK * HBLK, DP)
    W1p = jnp.pad(W1.reshape(5, D, HIDDEN), ((0, 0), (0, DP - D), (0, 0)))
    W1p = W1p.reshape(IN_DIM, HIDDEN)
    rows = _sc_gather(idx, table_p)
    x = rows.reshape(BATCH, IN_DIM)
    return _tc_mlp(x, W1p, b1, W2, b2)


# speedup vs baseline: 8.4724x; 1.0966x over previous
import jax
import jax.numpy as jnp
from jax import lax
from jax.experimental import pallas as pl
from jax.experimental.pallas import tpu as pltpu
from jax.experimental.pallas import tpu_sc as plsc

VOCAB = 1000000
D = 50
DP = 64
BATCH = 16384
IN_DIM = DP * 5
HIDDEN = 128
TAGS = 45

TOKENS = BATCH * 5
NC = 2
NS = 16
NW = NC * NS
R_PER_W = TOKENS // NW
CH = 128
N_CH = R_PER_W // CH

TBLK = 16384
HBLK = TBLK // 2
NBLK = pl.cdiv(VOCAB, TBLK)
PROWS = NBLK * TBLK


def _repack_kernel(t_ref, o_ref):
    xt = t_ref[...].T
    xp = jnp.concatenate(
        [xt, jnp.zeros((TBLK, DP - D), jnp.float32)], axis=1
    )
    o_ref[...] = jnp.concatenate([xp[:HBLK], xp[HBLK:]], axis=1)


def _tc_repack(tableT):
    return pl.pallas_call(
        _repack_kernel,
        grid=(NBLK,),
        in_specs=[pl.BlockSpec((D, TBLK), lambda i: (0, i))],
        out_specs=pl.BlockSpec((HBLK, 2 * DP), lambda i: (i, 0)),
        out_shape=jax.ShapeDtypeStruct((NBLK * HBLK, 2 * DP), jnp.float32),
    )(tableT)


def _sc_gather_kernel(idx_hbm, table_hbm, out_hbm, idx_v, rows_v, sem):
    wid = lax.axis_index("s") * NC + lax.axis_index("c")
    base = wid * R_PER_W
    pltpu.sync_copy(idx_hbm.at[wid], idx_v)

    def chunk(ci, carry):
        pltpu.async_copy(
            table_hbm.at[idx_v.at[ci]], rows_v, sem
        ).wait()
        pltpu.sync_copy(rows_v, out_hbm.at[pl.ds(base + ci * CH, CH)])
        return carry

    lax.fori_loop(0, N_CH, chunk, 0)


def _sc_gather(idx, table_p):
    mesh = plsc.VectorSubcoreMesh(core_axis_name="c", subcore_axis_name="s")
    return pl.kernel(
        _sc_gather_kernel,
        mesh=mesh,
        out_type=jax.ShapeDtypeStruct((TOKENS, DP), jnp.float32),
        scratch_types=[
            pltpu.VMEM((N_CH, CH), jnp.int32),
            pltpu.VMEM((CH, DP), jnp.float32),
            pltpu.SemaphoreType.DMA,
        ],
        compiler_params=pltpu.CompilerParams(use_tc_tiling_on_sc=False),
    )(idx.reshape(NW, N_CH, CH), table_p)


def _packed_row(idx):
    blk = idx // TBLK
    p = idx % TBLK
    return 2 * (blk * HBLK + (p % HBLK)) + p // HBLK


def _mlp_kernel(x_ref, w1_ref, b1_ref, w2_ref, b2_ref, o_ref):
    h = jnp.tanh(
        jnp.dot(x_ref[...], w1_ref[...], preferred_element_type=jnp.float32)
        + b1_ref[...]
    )
    o_ref[...] = (
        jnp.dot(h, w2_ref[...], preferred_element_type=jnp.float32)
        + b2_ref[...]
    )


def _tc_mlp(x, W1p, b1, W2, b2):
    blk = 2048
    grid = (BATCH // blk,)
    return pl.pallas_call(
        _mlp_kernel,
        grid=grid,
        in_specs=[
            pl.BlockSpec((blk, IN_DIM), lambda i: (i, 0)),
            pl.BlockSpec((IN_DIM, HIDDEN), lambda i: (0, 0)),
            pl.BlockSpec((1, HIDDEN), lambda i: (0, 0)),
            pl.BlockSpec((HIDDEN, TAGS), lambda i: (0, 0)),
            pl.BlockSpec((1, TAGS), lambda i: (0, 0)),
        ],
        out_specs=pl.BlockSpec((blk, TAGS), lambda i: (i, 0)),
        out_shape=jax.ShapeDtypeStruct((BATCH, TAGS), jnp.float32),
    )(x, W1p, b1.reshape(1, HIDDEN), W2, b2.reshape(1, TAGS))


def kernel(five_token_indices, table, W1, b1, W2, b2):
    idx = _packed_row(five_token_indices.reshape(-1).astype(jnp.int32))
    packed = _tc_repack(table.T)
    table_p = packed.reshape(2 * NBLK * HBLK, DP)
    W1p = jnp.pad(W1.reshape(5, D, HIDDEN), ((0, 0), (0, DP - D), (0, 0)))
    W1p = W1p.reshape(IN_DIM, HIDDEN)
    rows = _sc_gather(idx, table_p)
    x = rows.reshape(BATCH, IN_DIM)
    return _tc_mlp(x, W1p, b1, W2, b2)


# speedup vs baseline: 8.7219x; 1.0295x over previous
import jax
import jax.numpy as jnp
from jax import lax
from jax.experimental import pallas as pl
from jax.experimental.pallas import tpu as pltpu
from jax.experimental.pallas import tpu_sc as plsc

VOCAB = 1000000
D = 50
DP = 64
BATCH = 16384
IN_DIM = DP * 5
HIDDEN = 128
TAGS = 45

TOKENS = BATCH * 5
NC = 2
NS = 16
NW = NC * NS
R_PER_W = TOKENS // NW
CH = 128
N_CH = R_PER_W // CH

TBLK = 32768
HBLK = TBLK // 2
NBLK = pl.cdiv(VOCAB, TBLK)
PROWS = NBLK * TBLK


def _repack_kernel(t_ref, o_ref):
    xt = t_ref[...].T
    xp = jnp.concatenate(
        [xt, jnp.zeros((TBLK, DP - D), jnp.float32)], axis=1
    )
    o_ref[...] = jnp.concatenate([xp[:HBLK], xp[HBLK:]], axis=1)


def _tc_repack(tableT):
    return pl.pallas_call(
        _repack_kernel,
        grid=(NBLK,),
        in_specs=[pl.BlockSpec((D, TBLK), lambda i: (0, i))],
        out_specs=pl.BlockSpec((HBLK, 2 * DP), lambda i: (i, 0)),
        out_shape=jax.ShapeDtypeStruct((NBLK * HBLK, 2 * DP), jnp.float32),
    )(tableT)


def _sc_gather_kernel(idx_hbm, table_hbm, out_hbm, idx_v, rows_v, sem):
    wid = lax.axis_index("s") * NC + lax.axis_index("c")
    base = wid * R_PER_W
    pltpu.sync_copy(idx_hbm.at[wid], idx_v)

    def chunk(ci, carry):
        pltpu.async_copy(
            table_hbm.at[idx_v.at[ci]], rows_v, sem
        ).wait()
        pltpu.sync_copy(rows_v, out_hbm.at[pl.ds(base + ci * CH, CH)])
        return carry

    lax.fori_loop(0, N_CH, chunk, 0)


def _sc_gather(idx, table_p):
    mesh = plsc.VectorSubcoreMesh(core_axis_name="c", subcore_axis_name="s")
    return pl.kernel(
        _sc_gather_kernel,
        mesh=mesh,
        out_type=jax.ShapeDtypeStruct((TOKENS, DP), jnp.float32),
        scratch_types=[
            pltpu.VMEM((N_CH, CH), jnp.int32),
            pltpu.VMEM((CH, DP), jnp.float32),
            pltpu.SemaphoreType.DMA,
        ],
        compiler_params=pltpu.CompilerParams(use_tc_tiling_on_sc=False),
    )(idx.reshape(NW, N_CH, CH), table_p)


def _packed_row(idx):
    blk = idx // TBLK
    p = idx % TBLK
    return 2 * (blk * HBLK + (p % HBLK)) + p // HBLK


def _mlp_kernel(x_ref, w1_ref, b1_ref, w2_ref, b2_ref, o_ref):
    h = jnp.tanh(
        jnp.dot(x_ref[...], w1_ref[...], preferred_element_type=jnp.float32)
        + b1_ref[...]
    )
    o_ref[...] = (
        jnp.dot(h, w2_ref[...], preferred_element_type=jnp.float32)
        + b2_ref[...]
    )


def _tc_mlp(x, W1p, b1, W2, b2):
    blk = 2048
    grid = (BATCH // blk,)
    return pl.pallas_call(
        _mlp_kernel,
        grid=grid,
        in_specs=[
            pl.BlockSpec((blk, IN_DIM), lambda i: (i, 0)),
            pl.BlockSpec((IN_DIM, HIDDEN), lambda i: (0, 0)),
            pl.BlockSpec((1, HIDDEN), lambda i: (0, 0)),
            pl.BlockSpec((HIDDEN, TAGS), lambda i: (0, 0)),
            pl.BlockSpec((1, TAGS), lambda i: (0, 0)),
        ],
        out_specs=pl.BlockSpec((blk, TAGS), lambda i: (i, 0)),
        out_shape=jax.ShapeDtypeStruct((BATCH, TAGS), jnp.float32),
    )(x, W1p, b1.reshape(1, HIDDEN), W2, b2.reshape(1, TAGS))


def kernel(five_token_indices, table, W1, b1, W2, b2):
    idx = _packed_row(five_token_indices.reshape(-1).astype(jnp.int32))
    packed = _tc_repack(table.T)
    table_p = packed.reshape(2 * NBLK * HBLK, DP)
    W1p = jnp.pad(W1.reshape(5, D, HIDDEN), ((0, 0), (0, DP - D), (0, 0)))
    W1p = W1p.reshape(IN_DIM, HIDDEN)
    rows = _sc_gather(idx, table_p)
    x = rows.reshape(BATCH, IN_DIM)
    return _tc_mlp(x, W1p, b1, W2, b2)
